# baseline (speedup 1.0000x reference)
import functools

import jax
import jax.numpy as jnp
from jax import lax
from jax.experimental import pallas as pl
from jax.experimental.pallas import tpu as pltpu
from jax.experimental.pallas import tpu_sc as plsc

B, S, H, L = 16, 4096, 1024, 2048
NW = 32
K_SC = 8
K_TC = B - K_SC
TPB = NW // K_SC
IDX_PER_W = L // TPB
CH = 32
G = IDX_PER_W // CH
LANES = 16
HV = H // LANES
SCHUNK = 256
NCH = S // SCHUNK


def _pool_body(seg_hbm, hidden_hbm, out_hbm, idx_v, buf_a, buf_b, row_v,
               sem_a, sem_b):
    wid = lax.axis_index("s") * 2 + lax.axis_index("c")
    base = (wid // TPB) * S

    pltpu.sync_copy(seg_hbm.at[wid], idx_v)
    for g in range(G):
        for c in range(CH // LANES):
            sl = pl.ds(c * LANES, LANES)
            idx_v[g, sl] = idx_v[g, sl] + base

    for j in range(HV):
        row_v[pl.ds(j * LANES, LANES)] = jnp.zeros((LANES,), jnp.float32)

    bufs = (buf_a, buf_b)
    sems = (sem_a, sem_b)

    def _acc_from(buf):
        @plsc.parallel_loop(0, HV)
        def _(j):
            sl = pl.ds(j * LANES, LANES)
            v = buf[0, sl]
            for r in range(1, CH):
                v = v + buf[r, sl]
            row_v[sl] = row_v[sl] + v

    pending = [
        pltpu.async_copy(hidden_hbm.at[idx_v.at[0]], buf_a, sem_a),
        pltpu.async_copy(hidden_hbm.at[idx_v.at[1]], buf_b, sem_b),
    ]
    for g in range(G):
        p = g % 2
        pending[p].wait()
        _acc_from(bufs[p])
        if g + 2 < G:
            pending[p] = pltpu.async_copy(
                hidden_hbm.at[idx_v.at[g + 2]], bufs[p], sems[p])

    pltpu.sync_copy(row_v, out_hbm.at[wid])


_pool = functools.partial(
    pl.kernel,
    out_type=jax.ShapeDtypeStruct((NW, H), jnp.float32),
    mesh=plsc.VectorSubcoreMesh(core_axis_name="c", subcore_axis_name="s"),
    scratch_types=[
        pltpu.VMEM((G, CH), jnp.int32),
        pltpu.VMEM((CH, H), jnp.float32),
        pltpu.VMEM((CH, H), jnp.float32),
        pltpu.VMEM((H,), jnp.float32),
        pltpu.SemaphoreType.DMA,
        pltpu.SemaphoreType.DMA,
    ],
)(_pool_body)


def _matvec_body(idx_ref, hid_ref, o_ref):
    c = pl.program_id(1)
    idx_col = idx_ref[0]
    svals = lax.broadcasted_iota(jnp.int32, (1, SCHUNK), 1) + c * SCHUNK
    onehot = (idx_col == svals).astype(jnp.float32)
    ones = jnp.full((1, L), 1.0, dtype=jnp.float32)
    counts = lax.dot_general(ones, onehot, (((1,), (0,)), ((), ())),
                             preferred_element_type=jnp.float32)
    m = lax.dot_general(counts, hid_ref[0], (((1,), (0,)), ((), ())),
                        preferred_element_type=jnp.float32)

    @pl.when(c == 0)
    def _():
        o_ref[0] = m

    @pl.when(c != 0)
    def _():
        o_ref[0] = o_ref[0] + m


_matvec = functools.partial(
    pl.pallas_call,
    grid=(K_TC, NCH),
    in_specs=[
        pl.BlockSpec((1, L, 1), lambda b, c: (b + K_SC, 0, 0)),
        pl.BlockSpec((1, SCHUNK, H), lambda b, c: (b + K_SC, c, 0)),
    ],
    out_specs=pl.BlockSpec((1, 1, H), lambda b, c: (b, 0, 0)),
    out_shape=jax.ShapeDtypeStruct((K_TC, 1, H), jnp.float32),
)(_matvec_body)


def _final_body(p_ref, t_ref, w_ref, b_ref, o_ref):
    msc = p_ref[:, 0, :]
    for q in range(1, TPB):
        msc = msc + p_ref[:, q, :]
    mall = jnp.concatenate([msc, t_ref[:, 0, :]], axis=0) * (1.0 / L)
    y = lax.dot_general(mall, w_ref[...], (((1,), (1,)), ((), ())),
                        preferred_element_type=jnp.float32)
    o_ref[...] = jnp.tanh(y + b_ref[...])


def kernel(hidden_states, seg_indexs, W, b):
    seg = seg_indexs.astype(jnp.int32)
    seg_sc = seg[:K_SC].reshape(NW, G, CH)
    seg_tc = seg.reshape(B, L, 1)
    hidden_flat = hidden_states.reshape(B * S, H)
    partials = _pool(seg_sc, hidden_flat)
    m_tc = _matvec(seg_tc, hidden_states)
    return pl.pallas_call(
        _final_body,
        out_shape=jax.ShapeDtypeStruct((B, H), jnp.float32),
    )(partials.reshape(K_SC, TPB, H), m_tc, W, b.reshape(1, H))


# speedup vs baseline: 1.2152x; 1.2152x over previous
import functools

import jax
import jax.numpy as jnp
from jax import lax
from jax.experimental import pallas as pl
from jax.experimental.pallas import tpu as pltpu
from jax.experimental.pallas import tpu_sc as plsc

B, S, H, L = 16, 4096, 1024, 2048
NW = 32
K_SC = 8
K_TC = B - K_SC
TPB = NW // K_SC
IDX_PER_W = L // TPB
CH = 32
G = IDX_PER_W // CH
LANES = 16
HV = H // LANES
GH = IDX_PER_W // LANES
SCHUNK = 256
NCH = S // SCHUNK


IDXCH = 128
NIC = IDX_PER_W // IDXCH


BPC = K_TC // 2


def _hist_body(seg_hbm, cnt_hbm, idx_v, zer_v, zbuf_v, tbl):
    wid = lax.axis_index("c") * 16 + lax.axis_index("s")
    lb = (wid % 16) // TPB
    gb = wid // TPB

    pltpu.sync_copy(seg_hbm.at[wid], idx_v)
    base = lb * S
    for q in range(NIC):
        for c in range(IDXCH // LANES):
            sl = pl.ds(c * LANES, LANES)
            idx_v[q, sl] = idx_v[q, sl] + base
            zer_v[q, sl] = jnp.full((LANES,), 1.0, dtype=jnp.float32)

    @plsc.parallel_loop(0, S // LANES)
    def _(j):
        zbuf_v[pl.ds(j * LANES, LANES)] = jnp.zeros((LANES,), jnp.float32)

    @pl.when(wid % TPB == 0)
    def _():
        pltpu.sync_copy(zbuf_v, tbl.at[pl.ds(lb * S, S)])

    plsc.subcore_barrier()

    for q in range(NIC):
        pltpu.sync_copy(zer_v.at[q], tbl.at[idx_v.at[q]], add=True)

    plsc.subcore_barrier()

    @pl.when(wid % TPB == 0)
    def _():
        pltpu.sync_copy(tbl.at[pl.ds(lb * S, S)], cnt_hbm.at[gb])


_hist = functools.partial(
    pl.kernel,
    out_type=jax.ShapeDtypeStruct((K_TC, S), jnp.float32),
    mesh=plsc.VectorSubcoreMesh(core_axis_name="c", subcore_axis_name="s"),
    scratch_types=[
        pltpu.VMEM((NIC, IDXCH), jnp.int32),
        pltpu.VMEM((NIC, IDXCH), jnp.float32),
        pltpu.VMEM((S,), jnp.float32),
        pltpu.VMEM_SHARED((BPC * S,), jnp.float32),
    ],
)(_hist_body)


def _pool_body(seg_hbm, hidden_hbm, cnt_hbm, out_hbm, idx_v, buf_a, buf_b,
               row_v, sem_a, sem_b):
    del cnt_hbm
    wid = lax.axis_index("s") * 2 + lax.axis_index("c")
    base = (wid // TPB) * S

    pltpu.sync_copy(seg_hbm.at[wid], idx_v)
    for g in range(G):
        for c in range(CH // LANES):
            sl = pl.ds(c * LANES, LANES)
            idx_v[g, sl] = idx_v[g, sl] + base

    for j in range(HV):
        row_v[pl.ds(j * LANES, LANES)] = jnp.zeros((LANES,), jnp.float32)

    bufs = (buf_a, buf_b)
    sems = (sem_a, sem_b)

    def _acc_from(buf):
        @plsc.parallel_loop(0, HV)
        def _(j):
            sl = pl.ds(j * LANES, LANES)
            v = buf[0, sl]
            for r in range(1, CH):
                v = v + buf[r, sl]
            row_v[sl] = row_v[sl] + v

    pending = [
        pltpu.async_copy(hidden_hbm.at[idx_v.at[0]], buf_a, sem_a),
        pltpu.async_copy(hidden_hbm.at[idx_v.at[1]], buf_b, sem_b),
    ]
    for g in range(G):
        p = g % 2
        pending[p].wait()
        _acc_from(bufs[p])
        if g + 2 < G:
            pending[p] = pltpu.async_copy(
                hidden_hbm.at[idx_v.at[g + 2]], bufs[p], sems[p])

    pltpu.sync_copy(row_v, out_hbm.at[wid])


_pool = functools.partial(
    pl.kernel,
    out_type=jax.ShapeDtypeStruct((NW, H), jnp.float32),
    mesh=plsc.VectorSubcoreMesh(core_axis_name="c", subcore_axis_name="s"),
    scratch_types=[
        pltpu.VMEM((G, CH), jnp.int32),
        pltpu.VMEM((CH, H), jnp.float32),
        pltpu.VMEM((CH, H), jnp.float32),
        pltpu.VMEM((H,), jnp.float32),
        pltpu.SemaphoreType.DMA,
        pltpu.SemaphoreType.DMA,
    ],
)(_pool_body)


def _matvec_body(cnt_ref, hid_ref, o_ref):
    c = pl.program_id(1)
    counts = cnt_ref[0, 0]
    m = lax.dot_general(counts, hid_ref[0], (((1,), (0,)), ((), ())),
                        preferred_element_type=jnp.float32)

    @pl.when(c == 0)
    def _():
        o_ref[0] = m

    @pl.when(c != 0)
    def _():
        o_ref[0] = o_ref[0] + m


_matvec = functools.partial(
    pl.pallas_call,
    grid=(K_TC, NCH),
    in_specs=[
        pl.BlockSpec((1, 1, 1, SCHUNK), lambda b, c: (b, c, 0, 0)),
        pl.BlockSpec((1, SCHUNK, H), lambda b, c: (b + K_SC, c, 0)),
    ],
    out_specs=pl.BlockSpec((1, 1, H), lambda b, c: (b, 0, 0)),
    out_shape=jax.ShapeDtypeStruct((K_TC, 1, H), jnp.float32),
)(_matvec_body)


def _final_body(p_ref, t_ref, w_ref, b_ref, o_ref):
    msc = p_ref[:, 0, :]
    for q in range(1, TPB):
        msc = msc + p_ref[:, q, :]
    mall = jnp.concatenate([msc, t_ref[:, 0, :]], axis=0) * (1.0 / L)
    y = lax.dot_general(mall, w_ref[...], (((1,), (1,)), ((), ())),
                        preferred_element_type=jnp.float32)
    o_ref[...] = jnp.tanh(y + b_ref[...])


def kernel(hidden_states, seg_indexs, W, b):
    seg = seg_indexs.astype(jnp.int32)
    seg_sc = seg[:K_SC].reshape(NW, G, CH)
    seg_tc = seg[K_SC:].reshape(NW, NIC, IDXCH)
    hidden_flat = hidden_states.reshape(B * S, H)
    cnt = _hist(seg_tc)
    partials = _pool(seg_sc, hidden_flat, cnt)
    m_tc = _matvec(cnt.reshape(K_TC, NCH, 1, SCHUNK), hidden_states)
    return pl.pallas_call(
        _final_body,
        out_shape=jax.ShapeDtypeStruct((B, H), jnp.float32),
    )(partials.reshape(K_SC, TPB, H), m_tc, W, b.reshape(1, H))


# speedup vs baseline: 1.9542x; 1.6081x over previous
import functools

import jax
import jax.numpy as jnp
from jax import lax
from jax.experimental import pallas as pl
from jax.experimental.pallas import tpu as pltpu
from jax.experimental.pallas import tpu_sc as plsc

B, S, H, L = 16, 4096, 1024, 2048
NW = 32
IDX_PER_W = L * B // NW
CH = 16
NBUF = 4
G = IDX_PER_W // CH
LANES = 16
HV = H // LANES


def _pool_body(seg_hbm, hidden_hbm, out_hbm, idx_v, b0, b1, b2, b3, row_v,
               s0, s1, s2, s3):
    wid = lax.axis_index("s") * 2 + lax.axis_index("c")
    base = (wid // 2) * S

    pltpu.sync_copy(seg_hbm.at[wid], idx_v)
    for g in range(G):
        sl = pl.ds(0, LANES)
        idx_v[g, sl] = idx_v[g, sl] + base

    for j in range(HV):
        row_v[pl.ds(j * LANES, LANES)] = jnp.zeros((LANES,), jnp.float32)

    bufs = (b0, b1, b2, b3)
    sems = (s0, s1, s2, s3)

    def _acc_from(buf):
        @plsc.parallel_loop(0, HV)
        def _(j):
            sl = pl.ds(j * LANES, LANES)
            v = buf[0, sl]
            for r in range(1, CH):
                v = v + buf[r, sl]
            row_v[sl] = row_v[sl] + v

    pending = [
        pltpu.async_copy(hidden_hbm.at[idx_v.at[k]], bufs[k], sems[k])
        for k in range(NBUF)
    ]
    for g in range(G):
        p = g % NBUF
        pending[p].wait()
        _acc_from(bufs[p])
        if g + NBUF < G:
            pending[p] = pltpu.async_copy(
                hidden_hbm.at[idx_v.at[g + NBUF]], bufs[p], sems[p])

    pltpu.sync_copy(row_v, out_hbm.at[wid])


_pool = functools.partial(
    pl.kernel,
    out_type=jax.ShapeDtypeStruct((NW, H), jnp.float32),
    mesh=plsc.VectorSubcoreMesh(core_axis_name="c", subcore_axis_name="s"),
    scratch_types=[
        pltpu.VMEM((G, CH), jnp.int32),
        pltpu.VMEM((CH, H), jnp.float32),
        pltpu.VMEM((CH, H), jnp.float32),
        pltpu.VMEM((CH, H), jnp.float32),
        pltpu.VMEM((CH, H), jnp.float32),
        pltpu.VMEM((H,), jnp.float32),
        pltpu.SemaphoreType.DMA,
        pltpu.SemaphoreType.DMA,
        pltpu.SemaphoreType.DMA,
        pltpu.SemaphoreType.DMA,
    ],
)(_pool_body)


def _dense_body(p_ref, w_ref, b_ref, o_ref):
    m = (p_ref[:, 0, :] + p_ref[:, 1, :]) * (1.0 / L)
    y = lax.dot_general(m, w_ref[...], (((1,), (1,)), ((), ())),
                        preferred_element_type=jnp.float32)
    o_ref[...] = jnp.tanh(y + b_ref[...])


def kernel(hidden_states, seg_indexs, W, b):
    seg = seg_indexs.astype(jnp.int32).reshape(NW, G, CH)
    hidden_flat = hidden_states.reshape(B * S, H)
    partials = _pool(seg, hidden_flat)
    return pl.pallas_call(
        _dense_body,
        out_shape=jax.ShapeDtypeStruct((B, H), jnp.float32),
    )(partials.reshape(B, 2, H), W, b.reshape(1, H))


# speedup vs baseline: 2.2446x; 1.1486x over previous
import functools

import jax
import jax.numpy as jnp
from jax import lax
from jax.experimental import pallas as pl
from jax.experimental.pallas import tpu as pltpu
from jax.experimental.pallas import tpu_sc as plsc

B, S, H, L = 16, 4096, 1024, 2048
NW = 32
IDX_PER_W = L * B // NW
CH = 16
NBUF = 4
G = IDX_PER_W // CH
LANES = 16
HV = H // LANES


def _pool_body(seg_hbm, hidden_hbm, out_hbm, idx_v, b0, b1, b2, b3, row_v,
               s0, s1, s2, s3):
    wid = lax.axis_index("s") * 2 + lax.axis_index("c")
    base = (wid // 2) * S

    pltpu.sync_copy(seg_hbm.at[wid], idx_v)
    for g in range(G):
        sl = pl.ds(0, LANES)
        idx_v[g, sl] = idx_v[g, sl] + base

    for j in range(HV):
        row_v[pl.ds(j * LANES, LANES)] = jnp.zeros((LANES,), jnp.float32)

    bufs = (b0, b1, b2, b3)
    sems = (s0, s1, s2, s3)

    def _acc_from(buf):
        @plsc.parallel_loop(0, HV)
        def _(j):
            sl = pl.ds(j * LANES, LANES)
            v = buf[0, sl]
            for r in range(1, CH):
                v = v + buf[r, sl]
            row_v[sl] = row_v[sl] + v

    for k in range(NBUF):
        pltpu.async_copy(hidden_hbm.at[idx_v.at[k]], bufs[k], sems[k])

    @pl.loop(0, G // NBUF)
    def _(i):
        for k in range(NBUF):
            g = i * NBUF + k
            pltpu.make_async_copy(
                hidden_hbm.at[idx_v.at[k]], bufs[k], sems[k]).wait()
            _acc_from(bufs[k])

            @pl.when(g + NBUF < G)
            def _():
                pltpu.async_copy(
                    hidden_hbm.at[idx_v.at[g + NBUF]], bufs[k], sems[k])

    pltpu.sync_copy(row_v, out_hbm.at[wid])


_pool = functools.partial(
    pl.kernel,
    out_type=jax.ShapeDtypeStruct((NW, H), jnp.float32),
    mesh=plsc.VectorSubcoreMesh(core_axis_name="c", subcore_axis_name="s"),
    scratch_types=[
        pltpu.VMEM((G, CH), jnp.int32),
        pltpu.VMEM((CH, H), jnp.float32),
        pltpu.VMEM((CH, H), jnp.float32),
        pltpu.VMEM((CH, H), jnp.float32),
        pltpu.VMEM((CH, H), jnp.float32),
        pltpu.VMEM((H,), jnp.float32),
        pltpu.SemaphoreType.DMA,
        pltpu.SemaphoreType.DMA,
        pltpu.SemaphoreType.DMA,
        pltpu.SemaphoreType.DMA,
    ],
)(_pool_body)


def _dense_body(p_ref, w_ref, b_ref, o_ref):
    m = (p_ref[:, 0, :] + p_ref[:, 1, :]) * (1.0 / L)
    y = lax.dot_general(m, w_ref[...], (((1,), (1,)), ((), ())),
                        preferred_element_type=jnp.float32)
    o_ref[...] = jnp.tanh(y + b_ref[...])


def kernel(hidden_states, seg_indexs, W, b):
    seg = seg_indexs.astype(jnp.int32).reshape(NW, G, CH)
    hidden_flat = hidden_states.reshape(B * S, H)
    partials = _pool(seg, hidden_flat)
    return pl.pallas_call(
        _dense_body,
        out_shape=jax.ShapeDtypeStruct((B, H), jnp.float32),
    )(partials.reshape(B, 2, H), W, b.reshape(1, H))
